# baseline (speedup 1.0000x reference)
import jax
import jax.numpy as jnp
from jax.experimental import pallas as pl
from jax.experimental.pallas import tpu as pltpu

_BLOCK = 1024
_HALF = _BLOCK // 2


def _router_kernel(xa_ref, xb_ref, w_ref, o_ref):
    w = w_ref[...]
    for half, x_ref in ((0, xa_ref), (1, xb_ref)):
        logits = jax.lax.dot_general(
            x_ref[...],
            w,
            dimension_numbers=(((1,), (1,)), ((), ())),
            preferred_element_type=jnp.float32,
        )
        m = jnp.max(logits, axis=-1, keepdims=True)
        e = jnp.exp(logits - m)
        o_ref[pl.ds(half * _HALF, _HALF), :] = e / jnp.sum(
            e, axis=-1, keepdims=True
        )


def kernel(x, W):
    n_tokens, in_dim = x.shape
    n_experts = W.shape[0]
    return pl.pallas_call(
        _router_kernel,
        grid=(n_tokens // _BLOCK,),
        in_specs=[
            pl.BlockSpec((_HALF, in_dim), lambda i: (2 * i, 0)),
            pl.BlockSpec((_HALF, in_dim), lambda i: (2 * i + 1, 0)),
            pl.BlockSpec((n_experts, in_dim), lambda i: (0, 0)),
        ],
        out_specs=pl.BlockSpec((_BLOCK, n_experts), lambda i: (i, 0)),
        out_shape=jax.ShapeDtypeStruct((n_tokens, n_experts), jnp.float32),
        compiler_params=pltpu.CompilerParams(
            dimension_semantics=("arbitrary",)
        ),
    )(x, x, W)


# speedup vs baseline: 1.1888x; 1.1888x over previous
import jax
import jax.numpy as jnp
from jax.experimental import pallas as pl
from jax.experimental.pallas import tpu as pltpu

_BLOCK = 1024


def _router_kernel(x_ref, w_ref, o_ref):
    logits = jax.lax.dot_general(
        w_ref[...],
        x_ref[...],
        dimension_numbers=(((1,), (1,)), ((), ())),
        preferred_element_type=jnp.float32,
    )
    m = jnp.max(logits, axis=0, keepdims=True)
    e = jnp.exp(logits - m)
    o_ref[...] = e / jnp.sum(e, axis=0, keepdims=True)


def kernel(x, W):
    n_tokens, in_dim = x.shape
    n_experts = W.shape[0]
    out_t = pl.pallas_call(
        _router_kernel,
        grid=(n_tokens // _BLOCK,),
        in_specs=[
            pl.BlockSpec((_BLOCK, in_dim), lambda i: (i, 0)),
            pl.BlockSpec((n_experts, in_dim), lambda i: (0, 0)),
        ],
        out_specs=pl.BlockSpec((n_experts, _BLOCK), lambda i: (0, i)),
        out_shape=jax.ShapeDtypeStruct((n_experts, n_tokens), jnp.float32),
        compiler_params=pltpu.CompilerParams(
            dimension_semantics=("arbitrary",)
        ),
    )(x, W)
    return out_t.T
